# Initial kernel scaffold; baseline (speedup 1.0000x reference)
import jax
import jax.numpy as jnp
from jax.experimental import pallas as pl


def kernel(fea, params):
    raise NotImplementedError("write your pallas kernel here")


# speedup vs baseline: 1.4476x; 1.4476x over previous
import functools

import jax
import jax.numpy as jnp
import numpy as np
from jax.experimental import pallas as pl

_K = 16
_SPECS = [(3, 16, 24, 1), (24, 16, 32, 1), (32, 16, 48, 1), (48, 24, 48, 1),
          (48, 24, 64, 1), (64, 24, 64, 0), ('TD', 2048), (64, 24, 128, 1),
          (128, 32, 128, 0), ('TD', 1024), (128, 32, 256, 1),
          (256, 32, 256, 0), ('TD', 512), (256, 32, 512, 1),
          (512, 32, 512, 0), ('TD', 256), (512, 64, 1024, 1),
          (1024, 128, 1024, 0), ('TD', 128)]


def _aff(p):
    return p['W'], jnp.stack([p['b'], p['gamma'], p['beta']], axis=0)


def _apply_aff(h, aff):
    return (h + aff[0:1, :]) * aff[1:2, :] + aff[2:3, :]


def _leaky(x):
    return jnp.where(x >= 0, x, 0.2 * x)


def _knn_kernel(xq_ref, xa_ref, d_ref, i_ref, *, n, t):
    q = xq_ref[0]
    a = xa_ref[0]
    qsq = jnp.sum(q * q, axis=1, keepdims=True)
    asq = jnp.sum(a * a, axis=1, keepdims=True)
    prod = jax.lax.dot_general(q, a, (((1,), (1,)), ((), ())),
                               preferred_element_type=jnp.float32)
    d2 = qsq + asq.reshape(1, n) - 2.0 * prod
    dm = jnp.sqrt(jnp.maximum(d2, 0.0))
    iota = jax.lax.broadcasted_iota(jnp.int32, (t, n), 1)
    big = jnp.float32(np.inf)
    vals = []
    idxs = []
    for _ in range(_K):
        vmin = jnp.min(dm, axis=1, keepdims=True)
        cand = jnp.where(dm <= vmin, iota, n)
        imin = jnp.min(cand, axis=1, keepdims=True)
        vals.append(vmin)
        idxs.append(imin)
        dm = jnp.where(iota == imin, big, dm)
    d_ref[0] = jnp.concatenate(vals, axis=1)
    i_ref[0] = jnp.concatenate(idxs, axis=1)


def _knn(xyz):
    b, n, _ = xyz.shape
    t = min(n, 256)
    out = pl.pallas_call(
        functools.partial(_knn_kernel, n=n, t=t),
        grid=(b, n // t),
        in_specs=[
            pl.BlockSpec((1, t, 3), lambda bi, j: (bi, j, 0)),
            pl.BlockSpec((1, n, 3), lambda bi, j: (bi, 0, 0)),
        ],
        out_specs=[
            pl.BlockSpec((1, t, _K), lambda bi, j: (bi, j, 0)),
            pl.BlockSpec((1, t, _K), lambda bi, j: (bi, j, 0)),
        ],
        out_shape=[
            jax.ShapeDtypeStruct((b, n, _K), jnp.float32),
            jax.ShapeDtypeStruct((b, n, _K), jnp.int32),
        ],
    )(xyz, xyz)
    return out


def _lfa_kernel(gf_ref, orr_ref, ft_ref, rw_ref, ra_ref, aw_ref, ow_ref,
                oa_ref, sw_ref, sa_ref, out_ref):
    rw = rw_ref[...]
    ra = ra_ref[...]
    aw = aw_ref[...]
    cats = []
    logits = []
    for kk in range(_K):
        g = gf_ref[0, kk]
        o = orr_ref[0, kk]
        r = _leaky(_apply_aff(
            jnp.dot(o, rw, preferred_element_type=jnp.float32), ra))
        c = jnp.concatenate([g, r], axis=1)
        cats.append(c)
        logits.append(jnp.dot(c, aw, preferred_element_type=jnp.float32))
    m = logits[0]
    for l in logits[1:]:
        m = jnp.maximum(m, l)
    es = [jnp.exp(l - m) for l in logits]
    s = es[0]
    for e in es[1:]:
        s = s + e
    ws = [e / s for e in es]
    pooled = ws[0] * cats[0]
    for w, c in zip(ws[1:], cats[1:]):
        pooled = pooled + w * c
    ft = ft_ref[0]
    o = (_apply_aff(jnp.dot(pooled, ow_ref[...],
                            preferred_element_type=jnp.float32), oa_ref[...])
         + _apply_aff(jnp.dot(ft, sw_ref[...],
                              preferred_element_type=jnp.float32), sa_ref[...]))
    out_ref[0] = _leaky(o)


def _pick_t(n, d):
    if d >= 1024:
        t = 32
    elif d >= 512:
        t = 64
    else:
        t = 256
    return min(t, n)


def _lfa_pallas(feat, gf, orr, pr):
    b, n, c = feat.shape
    rw, ra = _aff(pr['rel'])
    aw = pr['attn_W']
    ow, oa = _aff(pr['out'])
    sw, sa = _aff(pr['short'])
    r = rw.shape[1]
    d = c + r
    cout = ow.shape[1]
    t = _pick_t(n, d)
    out = pl.pallas_call(
        _lfa_kernel,
        grid=(b, n // t),
        in_specs=[
            pl.BlockSpec((1, _K, t, c), lambda bi, j: (bi, 0, j, 0)),
            pl.BlockSpec((1, _K, t, 10), lambda bi, j: (bi, 0, j, 0)),
            pl.BlockSpec((1, t, c), lambda bi, j: (bi, j, 0)),
            pl.BlockSpec((10, r), lambda bi, j: (0, 0)),
            pl.BlockSpec((3, r), lambda bi, j: (0, 0)),
            pl.BlockSpec((d, d), lambda bi, j: (0, 0)),
            pl.BlockSpec((d, cout), lambda bi, j: (0, 0)),
            pl.BlockSpec((3, cout), lambda bi, j: (0, 0)),
            pl.BlockSpec((c, cout), lambda bi, j: (0, 0)),
            pl.BlockSpec((3, cout), lambda bi, j: (0, 0)),
        ],
        out_specs=pl.BlockSpec((1, t, cout), lambda bi, j: (bi, j, 0)),
        out_shape=jax.ShapeDtypeStruct((b, n, cout), jnp.float32),
    )(gf, orr, feat, rw, ra, aw, ow, oa, sw, sa)
    return out


def _gather_nb(x, idx_t):
    return jax.vmap(lambda xb, ib: xb[ib])(x, idx_t)


def _lfa(xyz, feat, orr, nidx_t, pr):
    if orr is None:
        dist, nidx = _knn(xyz)
        nidx_t = jnp.transpose(nidx, (0, 2, 1))
        nxyz = _gather_nb(xyz, nidx_t)
        dt = jnp.transpose(dist, (0, 2, 1))[..., None]
        ex = jnp.broadcast_to(xyz[:, None], nxyz.shape)
        orr = jnp.concatenate([dt, ex - nxyz, ex, nxyz], axis=-1)
    gf = _gather_nb(feat, nidx_t)
    out = _lfa_pallas(feat, gf, orr, pr)
    return out, orr, nidx_t


def _head_kernel(x_ref, w0_ref, a0_ref, w1_ref, a1_ref, o_ref, *, do_round):
    h = _leaky(_apply_aff(jnp.dot(x_ref[...], w0_ref[...],
                                  preferred_element_type=jnp.float32),
                          a0_ref[...]))
    h = _apply_aff(jnp.dot(h, w1_ref[...],
                           preferred_element_type=jnp.float32), a1_ref[...])
    if do_round:
        h = jnp.round(h)
    o_ref[...] = h


def _head(feat, p0, p1, do_round):
    b, n, c = feat.shape
    w0, b0 = _aff(p0)
    w1, b1 = _aff(p1)
    cout = w1.shape[1]
    x = feat.reshape(b * n, c)
    out = pl.pallas_call(
        functools.partial(_head_kernel, do_round=do_round),
        out_shape=jax.ShapeDtypeStruct((b * n, cout), jnp.float32),
    )(x, w0, b0, w1, b1)
    return out.reshape(b, n, cout)


def kernel(fea, params):
    b = fea.shape[0]
    xyz = fea[..., :3]
    feat = fea
    orr = None
    nidx_t = None
    li = 0
    for spec in _SPECS:
        if spec[0] == 'TD':
            n = spec[1]
            xyz = xyz[:, :n]
            feat = feat[:, :n]
        else:
            feat, orr, nidx_t = _lfa(xyz, feat, orr, nidx_t, params['enc'][li])
            li += 1
            if spec[3] == 0:
                orr = None
                nidx_t = None
    feat = _head(feat, params['enc_out0'], params['enc_out1'], do_round=True)
    feat, orr, nidx_t = _lfa(xyz, feat, None, None, params['dec'][0])
    feat, orr, nidx_t = _lfa(xyz, feat, orr, nidx_t, params['dec'][1])
    out = _head(feat, params['dec_out0'], params['dec_out1'], do_round=False)
    return out.reshape(b, 4096, 3)


# speedup vs baseline: 1.4559x; 1.0057x over previous
import functools

import jax
import jax.numpy as jnp
import numpy as np
from jax.experimental import pallas as pl

_K = 16
_SPECS = [(3, 16, 24, 1), (24, 16, 32, 1), (32, 16, 48, 1), (48, 24, 48, 1),
          (48, 24, 64, 1), (64, 24, 64, 0), ('TD', 2048), (64, 24, 128, 1),
          (128, 32, 128, 0), ('TD', 1024), (128, 32, 256, 1),
          (256, 32, 256, 0), ('TD', 512), (256, 32, 512, 1),
          (512, 32, 512, 0), ('TD', 256), (512, 64, 1024, 1),
          (1024, 128, 1024, 0), ('TD', 128)]


def _aff(p):
    return p['W'], jnp.stack([p['b'], p['gamma'], p['beta']], axis=0)


def _apply_aff(h, aff):
    return (h + aff[0:1, :]) * aff[1:2, :] + aff[2:3, :]


def _leaky(x):
    return jnp.where(x >= 0, x, 0.2 * x)


def _knn_kernel(xq_ref, xa_ref, d_ref, i_ref, *, n, t):
    q = xq_ref[0]
    a = xa_ref[0]
    qsq = jnp.sum(q * q, axis=1, keepdims=True)
    asq = jnp.sum(a * a, axis=1, keepdims=True)
    prod = jax.lax.dot_general(q, a, (((1,), (1,)), ((), ())),
                               preferred_element_type=jnp.float32)
    d2 = qsq + asq.reshape(1, n) - 2.0 * prod
    dm = jnp.sqrt(jnp.maximum(d2, 0.0))
    iota = jax.lax.broadcasted_iota(jnp.int32, (t, n), 1)
    big = jnp.float32(np.inf)
    vals = []
    idxs = []
    for _ in range(_K):
        vmin = jnp.min(dm, axis=1, keepdims=True)
        cand = jnp.where(dm <= vmin, iota, n)
        imin = jnp.min(cand, axis=1, keepdims=True)
        vals.append(vmin)
        idxs.append(imin)
        dm = jnp.where(iota == imin, big, dm)
    d_ref[0] = jnp.concatenate(vals, axis=1)
    i_ref[0] = jnp.concatenate(idxs, axis=1)


def _knn(xyz):
    b, n, _ = xyz.shape
    t = min(n, 512)
    out = pl.pallas_call(
        functools.partial(_knn_kernel, n=n, t=t),
        grid=(b, n // t),
        in_specs=[
            pl.BlockSpec((1, t, 3), lambda bi, j: (bi, j, 0)),
            pl.BlockSpec((1, n, 3), lambda bi, j: (bi, 0, 0)),
        ],
        out_specs=[
            pl.BlockSpec((1, t, _K), lambda bi, j: (bi, j, 0)),
            pl.BlockSpec((1, t, _K), lambda bi, j: (bi, j, 0)),
        ],
        out_shape=[
            jax.ShapeDtypeStruct((b, n, _K), jnp.float32),
            jax.ShapeDtypeStruct((b, n, _K), jnp.int32),
        ],
    )(xyz, xyz)
    return out


def _lfa_kernel(gf_ref, orr_ref, ft_ref, rw_ref, ra_ref, aw_ref, ow_ref,
                oa_ref, sw_ref, sa_ref, out_ref):
    rw = rw_ref[...]
    ra = ra_ref[...]
    aw = aw_ref[...]
    cats = []
    logits = []
    for kk in range(_K):
        g = gf_ref[0, kk]
        o = orr_ref[0, kk]
        r = _leaky(_apply_aff(
            jnp.dot(o, rw, preferred_element_type=jnp.float32), ra))
        c = jnp.concatenate([g, r], axis=1)
        cats.append(c)
        logits.append(jnp.dot(c, aw, preferred_element_type=jnp.float32))
    m = logits[0]
    for l in logits[1:]:
        m = jnp.maximum(m, l)
    es = [jnp.exp(l - m) for l in logits]
    s = es[0]
    for e in es[1:]:
        s = s + e
    ws = [e / s for e in es]
    pooled = ws[0] * cats[0]
    for w, c in zip(ws[1:], cats[1:]):
        pooled = pooled + w * c
    ft = ft_ref[0]
    o = (_apply_aff(jnp.dot(pooled, ow_ref[...],
                            preferred_element_type=jnp.float32), oa_ref[...])
         + _apply_aff(jnp.dot(ft, sw_ref[...],
                              preferred_element_type=jnp.float32), sa_ref[...]))
    out_ref[0] = _leaky(o)


def _pick_t(n, d):
    if d >= 1024:
        t = 64
    elif d >= 512:
        t = 128
    else:
        t = 512
    return min(t, n)


def _lfa_pallas(feat, gf, orr, pr):
    b, n, c = feat.shape
    rw, ra = _aff(pr['rel'])
    aw = pr['attn_W']
    ow, oa = _aff(pr['out'])
    sw, sa = _aff(pr['short'])
    r = rw.shape[1]
    d = c + r
    cout = ow.shape[1]
    t = _pick_t(n, d)
    out = pl.pallas_call(
        _lfa_kernel,
        grid=(b, n // t),
        in_specs=[
            pl.BlockSpec((1, _K, t, c), lambda bi, j: (bi, 0, j, 0)),
            pl.BlockSpec((1, _K, t, 10), lambda bi, j: (bi, 0, j, 0)),
            pl.BlockSpec((1, t, c), lambda bi, j: (bi, j, 0)),
            pl.BlockSpec((10, r), lambda bi, j: (0, 0)),
            pl.BlockSpec((3, r), lambda bi, j: (0, 0)),
            pl.BlockSpec((d, d), lambda bi, j: (0, 0)),
            pl.BlockSpec((d, cout), lambda bi, j: (0, 0)),
            pl.BlockSpec((3, cout), lambda bi, j: (0, 0)),
            pl.BlockSpec((c, cout), lambda bi, j: (0, 0)),
            pl.BlockSpec((3, cout), lambda bi, j: (0, 0)),
        ],
        out_specs=pl.BlockSpec((1, t, cout), lambda bi, j: (bi, j, 0)),
        out_shape=jax.ShapeDtypeStruct((b, n, cout), jnp.float32),
    )(gf, orr, feat, rw, ra, aw, ow, oa, sw, sa)
    return out


def _gather_nb(x, idx_t):
    return jax.vmap(lambda xb, ib: xb[ib])(x, idx_t)


def _lfa(xyz, feat, orr, nidx_t, pr):
    if orr is None:
        dist, nidx = _knn(xyz)
        nidx_t = jnp.transpose(nidx, (0, 2, 1))
        nxyz = _gather_nb(xyz, nidx_t)
        dt = jnp.transpose(dist, (0, 2, 1))[..., None]
        ex = jnp.broadcast_to(xyz[:, None], nxyz.shape)
        orr = jnp.concatenate([dt, ex - nxyz, ex, nxyz], axis=-1)
    gf = _gather_nb(feat, nidx_t)
    out = _lfa_pallas(feat, gf, orr, pr)
    return out, orr, nidx_t


def _head_kernel(x_ref, w0_ref, a0_ref, w1_ref, a1_ref, o_ref, *, do_round):
    h = _leaky(_apply_aff(jnp.dot(x_ref[...], w0_ref[...],
                                  preferred_element_type=jnp.float32),
                          a0_ref[...]))
    h = _apply_aff(jnp.dot(h, w1_ref[...],
                           preferred_element_type=jnp.float32), a1_ref[...])
    if do_round:
        h = jnp.round(h)
    o_ref[...] = h


def _head(feat, p0, p1, do_round):
    b, n, c = feat.shape
    w0, b0 = _aff(p0)
    w1, b1 = _aff(p1)
    cout = w1.shape[1]
    x = feat.reshape(b * n, c)
    out = pl.pallas_call(
        functools.partial(_head_kernel, do_round=do_round),
        out_shape=jax.ShapeDtypeStruct((b * n, cout), jnp.float32),
    )(x, w0, b0, w1, b1)
    return out.reshape(b, n, cout)


def kernel(fea, params):
    b = fea.shape[0]
    xyz = fea[..., :3]
    feat = fea
    orr = None
    nidx_t = None
    li = 0
    for spec in _SPECS:
        if spec[0] == 'TD':
            n = spec[1]
            xyz = xyz[:, :n]
            feat = feat[:, :n]
        else:
            feat, orr, nidx_t = _lfa(xyz, feat, orr, nidx_t, params['enc'][li])
            li += 1
            if spec[3] == 0:
                orr = None
                nidx_t = None
    feat = _head(feat, params['enc_out0'], params['enc_out1'], do_round=True)
    feat, orr, nidx_t = _lfa(xyz, feat, None, None, params['dec'][0])
    feat, orr, nidx_t = _lfa(xyz, feat, orr, nidx_t, params['dec'][1])
    out = _head(feat, params['dec_out0'], params['dec_out1'], do_round=False)
    return out.reshape(b, 4096, 3)


# speedup vs baseline: 1.4602x; 1.0030x over previous
import functools

import jax
import jax.numpy as jnp
import numpy as np
from jax.experimental import pallas as pl

_K = 16
_SPECS = [(3, 16, 24, 1), (24, 16, 32, 1), (32, 16, 48, 1), (48, 24, 48, 1),
          (48, 24, 64, 1), (64, 24, 64, 0), ('TD', 2048), (64, 24, 128, 1),
          (128, 32, 128, 0), ('TD', 1024), (128, 32, 256, 1),
          (256, 32, 256, 0), ('TD', 512), (256, 32, 512, 1),
          (512, 32, 512, 0), ('TD', 256), (512, 64, 1024, 1),
          (1024, 128, 1024, 0), ('TD', 128)]


def _aff(p):
    return p['W'], jnp.stack([p['b'], p['gamma'], p['beta']], axis=0)


def _apply_aff(h, aff):
    return (h + aff[0:1, :]) * aff[1:2, :] + aff[2:3, :]


def _leaky(x):
    return jnp.where(x >= 0, x, 0.2 * x)


def _knn_kernel(xq_ref, xa_ref, d_ref, i_ref, *, n, t):
    q = xq_ref[0]
    a = xa_ref[0]
    qsq = jnp.sum(q * q, axis=1, keepdims=True)
    asq = jnp.sum(a * a, axis=1, keepdims=True)
    prod = jax.lax.dot_general(q, a, (((1,), (1,)), ((), ())),
                               preferred_element_type=jnp.float32)
    d2 = qsq + asq.reshape(1, n) - 2.0 * prod
    dm = jnp.sqrt(jnp.maximum(d2, 0.0))
    iota = jax.lax.broadcasted_iota(jnp.int32, (t, n), 1)
    big = jnp.float32(np.inf)
    vals = []
    idxs = []
    for _ in range(_K):
        vmin = jnp.min(dm, axis=1, keepdims=True)
        cand = jnp.where(dm <= vmin, iota, n)
        imin = jnp.min(cand, axis=1, keepdims=True)
        vals.append(vmin)
        idxs.append(imin)
        dm = jnp.where(iota == imin, big, dm)
    d_ref[0] = jnp.concatenate(vals, axis=1)
    i_ref[0] = jnp.concatenate(idxs, axis=1)


def _knn(xyz):
    b, n, _ = xyz.shape
    t = min(n, 512)
    out = pl.pallas_call(
        functools.partial(_knn_kernel, n=n, t=t),
        grid=(b, n // t),
        in_specs=[
            pl.BlockSpec((1, t, 3), lambda bi, j: (bi, j, 0)),
            pl.BlockSpec((1, n, 3), lambda bi, j: (bi, 0, 0)),
        ],
        out_specs=[
            pl.BlockSpec((1, t, _K), lambda bi, j: (bi, j, 0)),
            pl.BlockSpec((1, t, _K), lambda bi, j: (bi, j, 0)),
        ],
        out_shape=[
            jax.ShapeDtypeStruct((b, n, _K), jnp.float32),
            jax.ShapeDtypeStruct((b, n, _K), jnp.int32),
        ],
    )(xyz, xyz)
    return out


def _lfa_kernel(gf_ref, orr_ref, ft_ref, rw_ref, ra_ref, aw_ref, ow_ref,
                oa_ref, sw_ref, sa_ref, out_ref):
    rw = rw_ref[...]
    ra = ra_ref[...]
    aw = aw_ref[...]
    t = ft_ref.shape[1]
    o_all = jnp.concatenate([orr_ref[0, kk] for kk in range(_K)], axis=0)
    g_all = jnp.concatenate([gf_ref[0, kk] for kk in range(_K)], axis=0)
    r_all = _leaky(_apply_aff(
        jnp.dot(o_all, rw, preferred_element_type=jnp.float32), ra))
    c_all = jnp.concatenate([g_all, r_all], axis=1)
    l_all = jnp.dot(c_all, aw, preferred_element_type=jnp.float32)
    cats = [c_all[kk * t:(kk + 1) * t] for kk in range(_K)]
    logits = [l_all[kk * t:(kk + 1) * t] for kk in range(_K)]
    m = logits[0]
    for l in logits[1:]:
        m = jnp.maximum(m, l)
    es = [jnp.exp(l - m) for l in logits]
    s = es[0]
    for e in es[1:]:
        s = s + e
    ws = [e / s for e in es]
    pooled = ws[0] * cats[0]
    for w, c in zip(ws[1:], cats[1:]):
        pooled = pooled + w * c
    ft = ft_ref[0]
    o = (_apply_aff(jnp.dot(pooled, ow_ref[...],
                            preferred_element_type=jnp.float32), oa_ref[...])
         + _apply_aff(jnp.dot(ft, sw_ref[...],
                              preferred_element_type=jnp.float32), sa_ref[...]))
    out_ref[0] = _leaky(o)


def _pick_t(n, d):
    if d >= 1024:
        t = 64
    elif d >= 512:
        t = 128
    else:
        t = 512
    return min(t, n)


def _lfa_pallas(feat, gf, orr, pr):
    b, n, c = feat.shape
    rw, ra = _aff(pr['rel'])
    aw = pr['attn_W']
    ow, oa = _aff(pr['out'])
    sw, sa = _aff(pr['short'])
    r = rw.shape[1]
    d = c + r
    cout = ow.shape[1]
    t = _pick_t(n, d)
    out = pl.pallas_call(
        _lfa_kernel,
        grid=(b, n // t),
        in_specs=[
            pl.BlockSpec((1, _K, t, c), lambda bi, j: (bi, 0, j, 0)),
            pl.BlockSpec((1, _K, t, 10), lambda bi, j: (bi, 0, j, 0)),
            pl.BlockSpec((1, t, c), lambda bi, j: (bi, j, 0)),
            pl.BlockSpec((10, r), lambda bi, j: (0, 0)),
            pl.BlockSpec((3, r), lambda bi, j: (0, 0)),
            pl.BlockSpec((d, d), lambda bi, j: (0, 0)),
            pl.BlockSpec((d, cout), lambda bi, j: (0, 0)),
            pl.BlockSpec((3, cout), lambda bi, j: (0, 0)),
            pl.BlockSpec((c, cout), lambda bi, j: (0, 0)),
            pl.BlockSpec((3, cout), lambda bi, j: (0, 0)),
        ],
        out_specs=pl.BlockSpec((1, t, cout), lambda bi, j: (bi, j, 0)),
        out_shape=jax.ShapeDtypeStruct((b, n, cout), jnp.float32),
    )(gf, orr, feat, rw, ra, aw, ow, oa, sw, sa)
    return out


def _gather_nb(x, idx_t):
    return jax.vmap(lambda xb, ib: xb[ib])(x, idx_t)


def _lfa(xyz, feat, orr, nidx_t, pr):
    if orr is None:
        dist, nidx = _knn(xyz)
        nidx_t = jnp.transpose(nidx, (0, 2, 1))
        nxyz = _gather_nb(xyz, nidx_t)
        dt = jnp.transpose(dist, (0, 2, 1))[..., None]
        ex = jnp.broadcast_to(xyz[:, None], nxyz.shape)
        orr = jnp.concatenate([dt, ex - nxyz, ex, nxyz], axis=-1)
    gf = _gather_nb(feat, nidx_t)
    out = _lfa_pallas(feat, gf, orr, pr)
    return out, orr, nidx_t


def _head_kernel(x_ref, w0_ref, a0_ref, w1_ref, a1_ref, o_ref, *, do_round):
    h = _leaky(_apply_aff(jnp.dot(x_ref[...], w0_ref[...],
                                  preferred_element_type=jnp.float32),
                          a0_ref[...]))
    h = _apply_aff(jnp.dot(h, w1_ref[...],
                           preferred_element_type=jnp.float32), a1_ref[...])
    if do_round:
        h = jnp.round(h)
    o_ref[...] = h


def _head(feat, p0, p1, do_round):
    b, n, c = feat.shape
    w0, b0 = _aff(p0)
    w1, b1 = _aff(p1)
    cout = w1.shape[1]
    x = feat.reshape(b * n, c)
    out = pl.pallas_call(
        functools.partial(_head_kernel, do_round=do_round),
        out_shape=jax.ShapeDtypeStruct((b * n, cout), jnp.float32),
    )(x, w0, b0, w1, b1)
    return out.reshape(b, n, cout)


def kernel(fea, params):
    b = fea.shape[0]
    xyz = fea[..., :3]
    feat = fea
    orr = None
    nidx_t = None
    li = 0
    for spec in _SPECS:
        if spec[0] == 'TD':
            n = spec[1]
            xyz = xyz[:, :n]
            feat = feat[:, :n]
        else:
            feat, orr, nidx_t = _lfa(xyz, feat, orr, nidx_t, params['enc'][li])
            li += 1
            if spec[3] == 0:
                orr = None
                nidx_t = None
    feat = _head(feat, params['enc_out0'], params['enc_out1'], do_round=True)
    feat, orr, nidx_t = _lfa(xyz, feat, None, None, params['dec'][0])
    feat, orr, nidx_t = _lfa(xyz, feat, orr, nidx_t, params['dec'][1])
    out = _head(feat, params['dec_out0'], params['dec_out1'], do_round=False)
    return out.reshape(b, 4096, 3)
